# Initial kernel scaffold; baseline (speedup 1.0000x reference)
import jax
import jax.numpy as jnp
from jax import lax
from jax.experimental import pallas as pl
from jax.experimental.pallas import tpu as pltpu
from jax.experimental.pallas import tpu_sc as plsc

N = 10000
E = 320000
DIM = 128

NC = 2
NS = 16
NW = NC * NS
K = 80
EPW = E // NW
ROWS_PER_W = EPW // K
NPAD = 10240
STRIPE = NPAD // NS


def _mesh():
    return plsc.VectorSubcoreMesh(core_axis_name="c", subcore_axis_name="s")


def _deg_body(dst_hbm, out_hbm, idx_v, ones_v, zero_v, deg_sh, sem):
    c = lax.axis_index("c")
    s = lax.axis_index("s")
    w = s * NC + c
    pltpu.sync_copy(dst_hbm.at[pl.ds(w * ROWS_PER_W, ROWS_PER_W)], idx_v)
    for i in range(K // 16):
        ones_v[pl.ds(i * 16, 16)] = jnp.ones((16,), jnp.float32)

    def zb(t, carry):
        zero_v[pl.ds(t * 16, 16)] = jnp.zeros((16,), jnp.float32)
        return carry

    lax.fori_loop(0, STRIPE // 16, zb, 0)
    pltpu.sync_copy(zero_v, deg_sh.at[pl.ds(s * STRIPE, STRIPE)])
    plsc.subcore_barrier()

    def step(j, carry):
        pltpu.sync_copy(ones_v, deg_sh.at[idx_v.at[j]], add=True)
        return carry

    lax.fori_loop(0, ROWS_PER_W, step, 0)
    plsc.subcore_barrier()
    pltpu.sync_copy(deg_sh.at[pl.ds(s * STRIPE, STRIPE)],
                    out_hbm.at[c, pl.ds(s * STRIPE, STRIPE)])


def _sc_degree(dst2d):
    kern = pl.kernel(
        _deg_body,
        out_type=jax.ShapeDtypeStruct((NC, NPAD), jnp.float32),
        mesh=_mesh(),
        scratch_types=[
            pltpu.VMEM((ROWS_PER_W, K), jnp.int32),
            pltpu.VMEM((K,), jnp.float32),
            pltpu.VMEM((STRIPE,), jnp.float32),
            pltpu.VMEM_SHARED((NPAD,), jnp.float32),
            pltpu.SemaphoreType.DMA,
        ],
    )
    return kern(dst2d)


def _msg_body(src_hbm, dst_hbm, hs_hbm, out_hbm, sidx, didx, rows, acc_sh, sem):
    c = lax.axis_index("c")
    s = lax.axis_index("s")
    w = s * NC + c
    pltpu.sync_copy(src_hbm.at[pl.ds(w * ROWS_PER_W, ROWS_PER_W)], sidx)
    pltpu.sync_copy(dst_hbm.at[pl.ds(w * ROWS_PER_W, ROWS_PER_W)], didx)

    def zr(t, carry):
        rows[t >> 3, pl.ds((t & 7) * 16, 16)] = jnp.zeros((16,), jnp.float32)
        return carry

    lax.fori_loop(0, K * (DIM // 16), zr, 0)
    for t in range(STRIPE // K):
        pltpu.sync_copy(rows, acc_sh.at[pl.ds(s * STRIPE + t * K, K)])
    plsc.subcore_barrier()

    def step(j, carry):
        pltpu.async_copy(hs_hbm.at[sidx.at[j]], rows, sem).wait()
        pltpu.sync_copy(rows, acc_sh.at[didx.at[j]], add=True)
        return carry

    lax.fori_loop(0, ROWS_PER_W, step, 0)
    plsc.subcore_barrier()
    pltpu.sync_copy(acc_sh.at[pl.ds(s * STRIPE, STRIPE)],
                    out_hbm.at[c, pl.ds(s * STRIPE, STRIPE)])


def _sc_edges(src2d, dst2d, hs):
    kern = pl.kernel(
        _msg_body,
        out_type=jax.ShapeDtypeStruct((NC, NPAD, DIM), jnp.float32),
        mesh=_mesh(),
        scratch_types=[
            pltpu.VMEM((ROWS_PER_W, K), jnp.int32),
            pltpu.VMEM((ROWS_PER_W, K), jnp.int32),
            pltpu.VMEM((K, DIM), jnp.float32),
            pltpu.VMEM_SHARED((NPAD, DIM), jnp.float32),
            pltpu.SemaphoreType.DMA,
        ],
    )
    return kern(src2d, dst2d, hs)


def _prep_body(x_ref, deg_ref, w_ref, g_ref, bt_ref, hs_ref):
    x = x_ref[...]
    mu = jnp.mean(x, axis=-1, keepdims=True)
    var = jnp.mean(x * x, axis=-1, keepdims=True) - mu * mu
    xr = (x - mu) * lax.rsqrt(var + 1e-5) * g_ref[...] + bt_ref[...]
    h = jnp.dot(xr, w_ref[...], preferred_element_type=jnp.float32)
    deg = jnp.sum(deg_ref[...], axis=-1, keepdims=True) + 1.0
    hs_ref[...] = h * lax.rsqrt(deg)


def _tc_prep(x, deg2, W, gamma, beta):
    B = 1000
    return pl.pallas_call(
        _prep_body,
        grid=(N // B,),
        in_specs=[
            pl.BlockSpec((B, DIM), lambda i: (i, 0)),
            pl.BlockSpec((B, 2), lambda i: (i, 0)),
            pl.BlockSpec((DIM, DIM), lambda i: (0, 0)),
            pl.BlockSpec((DIM,), lambda i: (0,)),
            pl.BlockSpec((DIM,), lambda i: (0,)),
        ],
        out_specs=pl.BlockSpec((B, DIM), lambda i: (i, 0)),
        out_shape=jax.ShapeDtypeStruct((N, DIM), jnp.float32),
    )(x, deg2, W, gamma, beta)


def _final_body(x_ref, hs_ref, acc_ref, deg_ref, b_ref, o_ref):
    q = acc_ref[0] + acc_ref[1] + hs_ref[...]
    deg = jnp.sum(deg_ref[...], axis=-1, keepdims=True) + 1.0
    agg = q * lax.rsqrt(deg)
    o_ref[...] = x_ref[...] + jnp.maximum(agg + b_ref[...], 0.0)


def _tc_final(x, hs, acc, deg2, b):
    B = 1000
    return pl.pallas_call(
        _final_body,
        grid=(N // B,),
        in_specs=[
            pl.BlockSpec((B, DIM), lambda i: (i, 0)),
            pl.BlockSpec((B, DIM), lambda i: (i, 0)),
            pl.BlockSpec((NC, B, DIM), lambda i: (0, i, 0)),
            pl.BlockSpec((B, 2), lambda i: (i, 0)),
            pl.BlockSpec((DIM,), lambda i: (0,)),
        ],
        out_specs=pl.BlockSpec((B, DIM), lambda i: (i, 0)),
        out_shape=jax.ShapeDtypeStruct((N, DIM), jnp.float32),
    )(x, hs, acc, deg2, b)


def kernel(x, edge_index, A_norm, edge_attr, W, b, gamma, beta):
    src2d = edge_index[0].reshape(E // K, K)
    dst2d = edge_index[1].reshape(E // K, K)
    deg_part = _sc_degree(dst2d)
    deg2 = deg_part[:, :N].T
    hs = _tc_prep(x, deg2, W, gamma, beta)
    acc = _sc_edges(src2d, dst2d, hs)
    x_out = _tc_final(x, hs, acc, deg2, b)
    return (x_out, edge_attr)


# speedup vs baseline: 23.3972x; 23.3972x over previous
import jax
import jax.numpy as jnp
from jax import lax
from jax.experimental import pallas as pl
from jax.experimental.pallas import tpu as pltpu
from jax.experimental.pallas import tpu_sc as plsc

N = 10000
E = 320000
DIM = 128

NC = 2
NS = 16
NW = NC * NS
K = 80
EPW = E // NW
ROWS_PER_W = EPW // K
NPAD = 10240
STRIPE = NPAD // NS


def _mesh():
    return plsc.VectorSubcoreMesh(core_axis_name="c", subcore_axis_name="s")


def _deg_body(dst_hbm, out_hbm, idx_v, ones_v, zero_v, deg_sh, sem):
    c = lax.axis_index("c")
    s = lax.axis_index("s")
    w = s * NC + c
    pltpu.sync_copy(dst_hbm.at[w], idx_v)
    for i in range(K // 16):
        ones_v[pl.ds(i * 16, 16)] = jnp.ones((16,), jnp.float32)

    def zb(t, carry):
        zero_v[pl.ds(t * 16, 16)] = jnp.zeros((16,), jnp.float32)
        return carry

    lax.fori_loop(0, STRIPE // 16, zb, 0)
    pltpu.sync_copy(zero_v, deg_sh.at[pl.ds(s * STRIPE, STRIPE)])
    plsc.subcore_barrier()

    def step(j, carry):
        pltpu.sync_copy(ones_v, deg_sh.at[idx_v.at[j]], add=True)
        return carry

    lax.fori_loop(0, ROWS_PER_W, step, 0)
    plsc.subcore_barrier()
    pltpu.sync_copy(deg_sh.at[pl.ds(s * STRIPE, STRIPE)],
                    out_hbm.at[c, 0, pl.ds(s * STRIPE, STRIPE)])


def _sc_degree(dst3d):
    kern = pl.kernel(
        _deg_body,
        out_type=jax.ShapeDtypeStruct((NC, 1, NPAD), jnp.float32),
        mesh=_mesh(),
        scratch_types=[
            pltpu.VMEM((ROWS_PER_W, K), jnp.int32),
            pltpu.VMEM((K,), jnp.float32),
            pltpu.VMEM((STRIPE,), jnp.float32),
            pltpu.VMEM_SHARED((NPAD,), jnp.float32),
            pltpu.SemaphoreType.DMA,
        ],
    )
    return kern(dst3d)


def _msg_body(src_hbm, dst_hbm, hs_hbm, out_hbm, sidx, didx, rows, acc_sh, sem):
    c = lax.axis_index("c")
    s = lax.axis_index("s")
    w = s * NC + c
    pltpu.sync_copy(src_hbm.at[w], sidx)
    pltpu.sync_copy(dst_hbm.at[w], didx)

    def zr(t, carry):
        rows[t >> 3, pl.ds((t & 7) * 16, 16)] = jnp.zeros((16,), jnp.float32)
        return carry

    lax.fori_loop(0, K * (DIM // 16), zr, 0)
    for t in range(STRIPE // K):
        pltpu.sync_copy(rows, acc_sh.at[pl.ds(s * STRIPE + t * K, K)])
    plsc.subcore_barrier()

    def step(j, carry):
        pltpu.async_copy(hs_hbm.at[sidx.at[j]], rows, sem).wait()
        pltpu.sync_copy(rows, acc_sh.at[didx.at[j]], add=True)
        return carry

    lax.fori_loop(0, ROWS_PER_W, step, 0)
    plsc.subcore_barrier()
    pltpu.sync_copy(acc_sh.at[pl.ds(s * STRIPE, STRIPE)],
                    out_hbm.at[c, pl.ds(s * STRIPE, STRIPE)])


def _sc_edges(src3d, dst3d, hs):
    kern = pl.kernel(
        _msg_body,
        out_type=jax.ShapeDtypeStruct((NC, NPAD, DIM), jnp.float32),
        mesh=_mesh(),
        scratch_types=[
            pltpu.VMEM((ROWS_PER_W, K), jnp.int32),
            pltpu.VMEM((ROWS_PER_W, K), jnp.int32),
            pltpu.VMEM((K, DIM), jnp.float32),
            pltpu.VMEM_SHARED((NPAD, DIM), jnp.float32),
            pltpu.SemaphoreType.DMA,
        ],
    )
    return kern(src3d, dst3d, hs)


def _prep_body(x_ref, deg_ref, w_ref, g_ref, bt_ref, hs_ref):
    x = x_ref[...]
    mu = jnp.mean(x, axis=-1, keepdims=True)
    var = jnp.mean(x * x, axis=-1, keepdims=True) - mu * mu
    xr = (x - mu) * lax.rsqrt(var + 1e-5) * g_ref[...] + bt_ref[...]
    h = jnp.dot(xr, w_ref[...], preferred_element_type=jnp.float32)
    deg = jnp.sum(deg_ref[...], axis=-1, keepdims=True) + 1.0
    hs_ref[...] = h * lax.rsqrt(deg)


def _tc_prep(x, deg2, W, gamma, beta):
    B = 1000
    return pl.pallas_call(
        _prep_body,
        grid=(N // B,),
        in_specs=[
            pl.BlockSpec((B, DIM), lambda i: (i, 0)),
            pl.BlockSpec((B, 2), lambda i: (i, 0)),
            pl.BlockSpec((DIM, DIM), lambda i: (0, 0)),
            pl.BlockSpec((DIM,), lambda i: (0,)),
            pl.BlockSpec((DIM,), lambda i: (0,)),
        ],
        out_specs=pl.BlockSpec((B, DIM), lambda i: (i, 0)),
        out_shape=jax.ShapeDtypeStruct((N, DIM), jnp.float32),
    )(x, deg2, W, gamma, beta)


def _final_body(x_ref, hs_ref, acc_ref, deg_ref, b_ref, o_ref):
    q = acc_ref[0] + acc_ref[1] + hs_ref[...]
    deg = jnp.sum(deg_ref[...], axis=-1, keepdims=True) + 1.0
    agg = q * lax.rsqrt(deg)
    o_ref[...] = x_ref[...] + jnp.maximum(agg + b_ref[...], 0.0)


def _tc_final(x, hs, acc, deg2, b):
    B = 1000
    return pl.pallas_call(
        _final_body,
        grid=(N // B,),
        in_specs=[
            pl.BlockSpec((B, DIM), lambda i: (i, 0)),
            pl.BlockSpec((B, DIM), lambda i: (i, 0)),
            pl.BlockSpec((NC, B, DIM), lambda i: (0, i, 0)),
            pl.BlockSpec((B, 2), lambda i: (i, 0)),
            pl.BlockSpec((DIM,), lambda i: (0,)),
        ],
        out_specs=pl.BlockSpec((B, DIM), lambda i: (i, 0)),
        out_shape=jax.ShapeDtypeStruct((N, DIM), jnp.float32),
    )(x, hs, acc, deg2, b)


def kernel(x, edge_index, A_norm, edge_attr, W, b, gamma, beta):
    src3d = edge_index[0].reshape(NW, ROWS_PER_W, K)
    dst3d = edge_index[1].reshape(NW, ROWS_PER_W, K)
    deg_part = _sc_degree(dst3d)
    deg2 = deg_part.reshape(NC, NPAD)[:, :N].T
    hs = _tc_prep(x, deg2, W, gamma, beta)
    acc = _sc_edges(src3d, dst3d, hs)
    x_out = _tc_final(x, hs, acc, deg2, b)
    return (x_out, edge_attr)
